# baseline (speedup 1.0000x reference)
import functools

import jax
import jax.numpy as jnp
from jax import lax
from jax.experimental import pallas as pl
from jax.experimental.pallas import tpu as pltpu
from jax.experimental.pallas import tpu_sc as plsc

B = 16384
D = 16
N = 5
OUT_D = 3 * D
NC = 2
NS = 16
NW = NC * NS
BPW = B // NW
CHUNK = 128
UCH = BPW // CHUNK

_MESH = plsc.VectorSubcoreMesh(core_axis_name="c", subcore_axis_name="s")


@functools.partial(
    pl.kernel,
    out_type=jax.ShapeDtypeStruct((OUT_D, B), jnp.float32),
    mesh=_MESH,
    compiler_params=pltpu.CompilerParams(use_tc_tiling_on_sc=False),
    scratch_types=[
        pltpu.VMEM((UCH, CHUNK), jnp.int32),
        pltpu.VMEM((N, UCH, CHUNK), jnp.int32),
        pltpu.VMEM((D, BPW), jnp.float32),
        pltpu.VMEM((N, D, BPW), jnp.float32),
        pltpu.VMEM((8, D), jnp.float32),
        pltpu.VMEM((D, BPW), jnp.float32),
        pltpu.VMEM((D, BPW), jnp.float32),
        pltpu.SemaphoreType.DMA,
    ],
)
def _sc_kernel(user_hbm, mem_hbm, utab_hbm, itab_hbm, w_hbm, out_hbm,
               uidx_v, midx_v, u_v, ir_v, w_v, up_v, drr_v, sem):
    wid = lax.axis_index("s") * NC + lax.axis_index("c")
    base = wid * BPW

    pltpu.sync_copy(user_hbm.at[wid], uidx_v)
    pltpu.sync_copy(mem_hbm.at[:, wid], midx_v)
    pltpu.sync_copy(w_hbm, w_v)

    def gather_k(k, carry):
        copies = []
        for c in range(UCH):
            copies.append(pltpu.async_copy(
                utab_hbm.at[k].at[uidx_v.at[c]],
                u_v.at[k, pl.ds(c * CHUNK, CHUNK)], sem))
        for n in range(N):
            for c in range(UCH):
                copies.append(pltpu.async_copy(
                    itab_hbm.at[k].at[midx_v.at[n, c]],
                    ir_v.at[n, k, pl.ds(c * CHUNK, CHUNK)], sem))
        for cp in copies:
            cp.wait()
        return carry

    lax.fori_loop(0, D, gather_k, 0)

    w0 = w_v[0]
    w1 = w_v[1]
    w2 = w_v[2]
    w3 = w_v[3]
    w4 = w_v[4]
    cb = w_v[5]

    def body(i, carry):
        k = i // (BPW // 16)
        j = (i % (BPW // 16)) * 16
        sl = pl.ds(j, 16)
        drr = (cb + w0 * ir_v[0, k, sl] + w1 * ir_v[1, k, sl]
               + w2 * ir_v[2, k, sl] + w3 * ir_v[3, k, sl]
               + w4 * ir_v[4, k, sl])
        up_v[k, sl] = u_v[k, sl] * drr
        drr_v[k, sl] = drr
        return carry

    lax.fori_loop(0, D * (BPW // 16), body, 0)

    pltpu.sync_copy(u_v, out_hbm.at[pl.ds(0, D), pl.ds(base, BPW)])
    pltpu.sync_copy(up_v, out_hbm.at[pl.ds(D, D), pl.ds(base, BPW)])
    pltpu.sync_copy(drr_v, out_hbm.at[pl.ds(2 * D, D), pl.ds(base, BPW)])


def kernel(user, memory, user_table, item_table, conv_w, conv_b):
    user2d = user.astype(jnp.int32).reshape(NW, UCH, CHUNK)
    mem_t = memory.astype(jnp.int32).T.reshape(N, NW, UCH, CHUNK)
    ut_t = user_table.T
    it_t = item_table.T
    w = conv_w[0, :, 0]
    wpad = jnp.zeros((8,), jnp.float32).at[:N].set(w).at[N].set(conv_b[0])
    w_rows = jnp.broadcast_to(wpad[:, None], (8, D))
    out_t = _sc_kernel(user2d, mem_t, ut_t, it_t, w_rows)
    return out_t.T


# speedup vs baseline: 3.2158x; 3.2158x over previous
import functools

import jax
import jax.numpy as jnp
from jax import lax
from jax.experimental import pallas as pl
from jax.experimental.pallas import tpu as pltpu
from jax.experimental.pallas import tpu_sc as plsc

B = 16384
D = 16
N = 5
OUT_D = 3 * D
NC = 2
NS = 16
NW = NC * NS
BPW = B // NW
CHUNK = 128
UCH = BPW // CHUNK
ICH = (BPW * N) // CHUNK

_MESH = plsc.VectorSubcoreMesh(core_axis_name="c", subcore_axis_name="s")


@functools.partial(
    pl.kernel,
    out_type=jax.ShapeDtypeStruct((B, OUT_D), jnp.float32),
    mesh=_MESH,
    compiler_params=pltpu.CompilerParams(use_tc_tiling_on_sc=False),
    scratch_types=[
        pltpu.VMEM((UCH, CHUNK), jnp.int32),
        pltpu.VMEM((ICH, CHUNK), jnp.int32),
        pltpu.VMEM((BPW, D), jnp.float32),
        pltpu.VMEM((BPW * N, D), jnp.float32),
        pltpu.VMEM((8, D), jnp.float32),
        pltpu.VMEM((BPW, OUT_D), jnp.float32),
        pltpu.SemaphoreType.DMA,
        pltpu.SemaphoreType.DMA,
    ],
)
def _sc_kernel(user_hbm, mem_hbm, utab_hbm, itab_hbm, w_hbm, out_hbm,
               uidx_v, midx_v, urows_v, irows_v, w_v, out_v, sem_u, sem_i):
    wid = lax.axis_index("s") * NC + lax.axis_index("c")
    base = wid * BPW

    pltpu.sync_copy(user_hbm.at[wid], uidx_v)
    pltpu.sync_copy(mem_hbm.at[wid], midx_v)
    pltpu.sync_copy(w_hbm, w_v)

    copies = []
    for j in range(UCH):
        copies.append(pltpu.async_copy(
            utab_hbm.at[uidx_v.at[j]],
            urows_v.at[pl.ds(j * CHUNK, CHUNK)], sem_u))
    for j in range(ICH):
        copies.append(pltpu.async_copy(
            itab_hbm.at[midx_v.at[j]],
            irows_v.at[pl.ds(j * CHUNK, CHUNK)], sem_i))
    for cp in copies:
        cp.wait()

    w0 = w_v[0]
    w1 = w_v[1]
    w2 = w_v[2]
    w3 = w_v[3]
    w4 = w_v[4]
    cb = w_v[5]

    def body(b, carry):
        ib = b * N
        drr = (cb + w0 * irows_v[ib] + w1 * irows_v[ib + 1]
               + w2 * irows_v[ib + 2] + w3 * irows_v[ib + 3]
               + w4 * irows_v[ib + 4])
        u = urows_v[b]
        out_v[b, pl.ds(0, D)] = u
        out_v[b, pl.ds(D, D)] = u * drr
        out_v[b, pl.ds(2 * D, D)] = drr
        return carry

    lax.fori_loop(0, BPW, body, 0)

    pltpu.sync_copy(out_v, out_hbm.at[pl.ds(base, BPW)])


def kernel(user, memory, user_table, item_table, conv_w, conv_b):
    user2d = user.astype(jnp.int32).reshape(NW, UCH, CHUNK)
    mem2d = memory.astype(jnp.int32).reshape(NW, ICH, CHUNK)
    w = conv_w[0, :, 0]
    wpad = jnp.zeros((8,), jnp.float32).at[:N].set(w).at[N].set(conv_b[0])
    w_rows = jnp.broadcast_to(wpad[:, None], (8, D))
    return _sc_kernel(user2d, mem2d, user_table, item_table, w_rows)
